# Initial kernel scaffold; baseline (speedup 1.0000x reference)
import jax
import jax.numpy as jnp
from jax.experimental import pallas as pl


def kernel(x, edge_index0, edge_index1, W0, al0, ar0, W1, al1, ar1, sw1, sb1, sw2, cw, cb):
    raise NotImplementedError("write your pallas kernel here")


# speedup vs baseline: 23.2101x; 23.2101x over previous
import functools

import jax
import jax.numpy as jnp
from jax import lax
from jax.experimental import pallas as pl
from jax.experimental.pallas import tpu as pltpu
from jax.experimental.pallas import tpu_sc as plsc

_N = 10000
_E = 160000
_IN = 128
_H = 8
_F = 32
_D = 256
_HID = 128
_OUT = 2

_NT = 16
_RA = 624
_RB = 640
_EPT = _E // _NT
_CH = 80
_NCH = _EPT // _CH

_BLK = 400
_NB = _N // _BLK


def _elu(v):
    return jnp.where(v > 0.0, v, jnp.exp(jnp.minimum(v, 0.0)) - 1.0)


def _tc1_body(x_r, w_r, m_r, fa0, fb0, fa1, fb1, e0, e1):
    f01 = jnp.dot(x_r[...], w_r[...], preferred_element_type=jnp.float32)
    ee = jnp.dot(f01, m_r[...], preferred_element_type=jnp.float32)
    fa0[...] = f01[:, 0:128]
    fb0[...] = f01[:, 128:256]
    fa1[...] = f01[:, 256:384]
    fb1[...] = f01[:, 384:512]
    pad = jnp.zeros((_BLK, 96), jnp.float32)
    e0[...] = jnp.concatenate(
        [ee[:, 0:8], ee[:, 0:8], ee[:, 8:16], ee[:, 8:16], pad], axis=1)
    e1[...] = jnp.concatenate(
        [ee[:, 16:24], ee[:, 16:24], ee[:, 24:32], ee[:, 24:32], pad], axis=1)


def _tc_project(x, wcat, m):
    in_specs = [
        pl.BlockSpec((_BLK, _IN), lambda i: (i, 0)),
        pl.BlockSpec((_IN, 2 * _D), lambda i: (0, 0)),
        pl.BlockSpec((2 * _D, 4 * _H), lambda i: (0, 0)),
    ]
    out_shape = [jax.ShapeDtypeStruct((_N, 128), jnp.float32)] * 6
    out_specs = [pl.BlockSpec((_BLK, 128), lambda i: (i, 0))] * 6
    return pl.pallas_call(_tc1_body, grid=(_NB,), in_specs=in_specs,
                          out_specs=out_specs, out_shape=out_shape)(x, wcat, m)


def _sc_softmax_denom(src0, dst0, src1, dst1, e0, e1, zrow):
    mesh = plsc.VectorSubcoreMesh(core_axis_name="c", subcore_axis_name="s")

    @functools.partial(
        pl.kernel,
        mesh=mesh,
        out_type=[
            jax.ShapeDtypeStruct((_E, 16), jnp.float32),
            jax.ShapeDtypeStruct((_E, 16), jnp.float32),
            jax.ShapeDtypeStruct((_N, 128), jnp.float32),
            jax.ShapeDtypeStruct((_N, 128), jnp.float32),
        ],
        scratch_types=[
            pltpu.VMEM((1, _CH), jnp.int32),
            pltpu.VMEM((1, _CH), jnp.int32),
            pltpu.VMEM((_CH, 128), jnp.float32),
            pltpu.VMEM((_CH, 128), jnp.float32),
            pltpu.VMEM((_CH, 128), jnp.float32),
            pltpu.VMEM((_CH, 16), jnp.float32),
            pltpu.VMEM_SHARED((_N, 128), jnp.float32),
            pltpu.SemaphoreType.DMA,
        ],
    )
    def k(src0_h, dst0_h, src1_h, dst1_h, e0_h, e1_h, z_h,
          ex0_h, ex1_h, s0_h, s1_h,
          src_v, dst_v, a_v, b_v, ex_v, ex_s, s_acc, sem):
        cid = lax.axis_index("c")
        sid = lax.axis_index("s")

        @pl.when(sid < 15)
        def _z0():
            pltpu.sync_copy(z_h.at[pl.ds(0, _RA)],
                            s_acc.at[pl.ds(sid * _RA, _RA)])

        @pl.when(sid == 15)
        def _z1():
            pltpu.sync_copy(z_h, s_acc.at[pl.ds(15 * _RA, _RB)])

        def zrow_(j, c2):
            for t in range(7):
                ex_v[j, pl.ds(16 + 16 * t, 16)] = jnp.zeros((16,), jnp.float32)
            return c2

        lax.fori_loop(0, _CH, zrow_, 0)
        plsc.subcore_barrier()

        def run(src_h, dst_h, e_h, ex_h):
            def chunk(i, carry):
                base = sid * _EPT + i * _CH
                pltpu.sync_copy(src_h.at[pl.ds(base, _CH)], src_v.at[0])
                pltpu.sync_copy(dst_h.at[pl.ds(base, _CH)], dst_v.at[0])
                pltpu.async_copy(e_h.at[src_v.at[0]], a_v, sem).wait()
                pltpu.async_copy(e_h.at[dst_v.at[0]], b_v, sem).wait()

                def row(j, c2):
                    e = a_v[j, pl.ds(0, 16)] + b_v[j, pl.ds(16, 16)]
                    e = jnp.where(e >= 0.0, e, 0.2 * e)
                    xv = jnp.exp(e)
                    ex_v[j, pl.ds(0, 16)] = xv
                    ex_s[j] = xv
                    return c2

                lax.fori_loop(0, _CH, row, 0)
                pltpu.sync_copy(ex_s, ex_h.at[pl.ds(base, _CH)])
                pltpu.sync_copy(ex_v, s_acc.at[dst_v.at[0]], add=True)
                return carry

            lax.fori_loop(0, _NCH, chunk, 0)

        @pl.when(cid == 0)
        def _p0():
            run(src0_h, dst0_h, e0_h, ex0_h)

        @pl.when(cid == 1)
        def _p1():
            run(src1_h, dst1_h, e1_h, ex1_h)

        plsc.subcore_barrier()

        def flush(dst_tab):
            @pl.when(sid < 15)
            def _o0():
                pltpu.sync_copy(s_acc.at[pl.ds(sid * _RA, _RA)],
                                dst_tab.at[pl.ds(sid * _RA, _RA)])

            @pl.when(sid == 15)
            def _o1():
                pltpu.sync_copy(s_acc.at[pl.ds(15 * _RA, _RB)],
                                dst_tab.at[pl.ds(15 * _RA, _RB)])

        @pl.when(cid == 0)
        def _f0():
            flush(s0_h)

        @pl.when(cid == 1)
        def _f1():
            flush(s1_h)

    return k(src0, dst0, src1, dst1, e0, e1, zrow)


def _sc_aggregate(src0, dst0, src1, dst1, ex0, ex1, s0, s1,
                  fa0, fb0, fa1, fb1, zblk):
    mesh = plsc.VectorSubcoreMesh(core_axis_name="c", subcore_axis_name="s")

    @functools.partial(
        pl.kernel,
        mesh=mesh,
        out_type=[jax.ShapeDtypeStruct((_N, 128), jnp.float32)] * 4,
        scratch_types=[
            pltpu.VMEM((1, _CH), jnp.int32),
            pltpu.VMEM((1, _CH), jnp.int32),
            pltpu.VMEM((_CH, 128), jnp.float32),
            pltpu.VMEM((_CH, 16), jnp.float32),
            pltpu.VMEM((16, 128), jnp.float32),
            pltpu.VMEM((16, 128), jnp.float32),
            pltpu.VMEM_SHARED((_N, 128), jnp.float32),
            pltpu.SemaphoreType.DMA,
        ],
    )
    def k(src0_h, dst0_h, src1_h, dst1_h, ex0_h, ex1_h, s0_h, s1_h,
          fa0_h, fb0_h, fa1_h, fb1_h, z_h,
          oa0_h, ob0_h, oa1_h, ob1_h,
          src_v, dst_v, f_v, ex_v, g_v, s_t, acc, sem):
        cid = lax.axis_index("c")
        sid = lax.axis_index("s")

        def run(src_h, dst_h, ex_h, s_h, feat_h, out_h, hoff):
            @pl.when(sid < 15)
            def _z0():
                pltpu.sync_copy(z_h.at[pl.ds(0, _RA)],
                                acc.at[pl.ds(sid * _RA, _RA)])

            @pl.when(sid == 15)
            def _z1():
                pltpu.sync_copy(z_h, acc.at[pl.ds(15 * _RA, _RB)])

            plsc.subcore_barrier()

            def chunk(i, carry):
                base = sid * _EPT + i * _CH
                pltpu.sync_copy(src_h.at[pl.ds(base, _CH)], src_v.at[0])
                pltpu.sync_copy(dst_h.at[pl.ds(base, _CH)], dst_v.at[0])
                pltpu.async_copy(feat_h.at[src_v.at[0]], f_v, sem).wait()
                pltpu.sync_copy(ex_h.at[pl.ds(base, _CH)], ex_v)

                def row(j, c2):
                    av = ex_v[j]
                    for kk in range(4):
                        a = av[hoff + kk]
                        c0 = 32 * kk
                        f_v[j, pl.ds(c0, 16)] = f_v[j, pl.ds(c0, 16)] * a
                        f_v[j, pl.ds(c0 + 16, 16)] = (
                            f_v[j, pl.ds(c0 + 16, 16)] * a)
                    return c2

                lax.fori_loop(0, _CH, row, 0)
                pltpu.sync_copy(f_v, acc.at[dst_v.at[0]], add=True)
                return carry

            lax.fori_loop(0, _NCH, chunk, 0)
            plsc.subcore_barrier()

            def flushchunk(r, carry):
                r0 = sid * _RA + r * 16
                pltpu.sync_copy(acc.at[pl.ds(r0, 16)], g_v)
                pltpu.sync_copy(s_h.at[pl.ds(r0, 16)], s_t)

                def row(j, c2):
                    sv = s_t[j, pl.ds(0, 16)] + 1e-9
                    for kk in range(4):
                        d = sv[hoff + kk]
                        c0 = 32 * kk
                        g_v[j, pl.ds(c0, 16)] = g_v[j, pl.ds(c0, 16)] / d
                        g_v[j, pl.ds(c0 + 16, 16)] = (
                            g_v[j, pl.ds(c0 + 16, 16)] / d)
                    return c2

                lax.fori_loop(0, 16, row, 0)
                pltpu.sync_copy(g_v, out_h.at[pl.ds(r0, 16)])
                return carry

            nfc = jnp.where(sid == 15, _RB // 16, _RA // 16)
            lax.fori_loop(0, nfc, flushchunk, 0)

        @pl.when(cid == 0)
        def _a():
            run(src0_h, dst0_h, ex0_h, s0_h, fa0_h, oa0_h, 0)
            run(src1_h, dst1_h, ex1_h, s1_h, fa1_h, oa1_h, 0)

        @pl.when(cid == 1)
        def _b():
            run(src0_h, dst0_h, ex0_h, s0_h, fb0_h, ob0_h, 4)
            run(src1_h, dst1_h, ex1_h, s1_h, fb1_h, ob1_h, 4)

    return k(src0, dst0, src1, dst1, ex0, ex1, s0, s1,
             fa0, fb0, fa1, fb1, zblk)


def _tc2a_body(a0, b0, a1, b1, sw1_r, sb1_r, sw2_r, cw_r, lp0_r, lp1_r, w_r):
    z0 = jnp.concatenate([_elu(a0[...]), _elu(b0[...])], axis=1)
    z1 = jnp.concatenate([_elu(a1[...]), _elu(b1[...])], axis=1)
    t0 = jnp.tanh(jnp.dot(z0, sw1_r[...], preferred_element_type=jnp.float32)
                  + sb1_r[...])
    t1 = jnp.tanh(jnp.dot(z1, sw1_r[...], preferred_element_type=jnp.float32)
                  + sb1_r[...])
    wp0 = jnp.sum(t0 * sw2_r[...])
    wp1 = jnp.sum(t1 * sw2_r[...])
    lp0_r[...] = jnp.dot(z0, cw_r[...], preferred_element_type=jnp.float32)
    lp1_r[...] = jnp.dot(z1, cw_r[...], preferred_element_type=jnp.float32)

    @pl.when(pl.program_id(0) == 0)
    def _init():
        w_r[...] = jnp.zeros_like(w_r)

    w_r[...] += jnp.concatenate([wp0.reshape(1, 1), wp1.reshape(1, 1)], axis=1)


def _tc_pool(oa0, ob0, oa1, ob1, sw1, sb1_row, sw2_row, cw):
    in_specs = ([pl.BlockSpec((_BLK, 128), lambda i: (i, 0))] * 4 + [
        pl.BlockSpec((_D, _HID), lambda i: (0, 0)),
        pl.BlockSpec((1, _HID), lambda i: (0, 0)),
        pl.BlockSpec((1, _HID), lambda i: (0, 0)),
        pl.BlockSpec((_D, _OUT), lambda i: (0, 0)),
    ])
    out_shape = [
        jax.ShapeDtypeStruct((_N, _OUT), jnp.float32),
        jax.ShapeDtypeStruct((_N, _OUT), jnp.float32),
        jax.ShapeDtypeStruct((1, 2), jnp.float32),
    ]
    out_specs = [
        pl.BlockSpec((_BLK, _OUT), lambda i: (i, 0)),
        pl.BlockSpec((_BLK, _OUT), lambda i: (i, 0)),
        pl.BlockSpec((1, 2), lambda i: (0, 0)),
    ]
    return pl.pallas_call(_tc2a_body, grid=(_NB,), in_specs=in_specs,
                          out_specs=out_specs, out_shape=out_shape)(
        oa0, ob0, oa1, ob1, sw1, sb1_row, sw2_row, cw)


def _tc2b_body(lp0_r, lp1_r, w_r, cb_r, out_r):
    wm = w_r[...] / float(_N)
    ew = jnp.exp(wm - jnp.max(wm))
    beta = ew / jnp.sum(ew)
    out_r[...] = (lp0_r[...] * beta[:, 0:1] + lp1_r[...] * beta[:, 1:2]
                  + cb_r[...])


def _tc_combine(lp0, lp1, w, cb_row):
    in_specs = [
        pl.BlockSpec((_BLK, _OUT), lambda i: (i, 0)),
        pl.BlockSpec((_BLK, _OUT), lambda i: (i, 0)),
        pl.BlockSpec((1, 2), lambda i: (0, 0)),
        pl.BlockSpec((1, _OUT), lambda i: (0, 0)),
    ]
    return pl.pallas_call(
        _tc2b_body, grid=(_NB,), in_specs=in_specs,
        out_specs=pl.BlockSpec((_BLK, _OUT), lambda i: (i, 0)),
        out_shape=jax.ShapeDtypeStruct((_N, _OUT), jnp.float32),
    )(lp0, lp1, w, cb_row)


def kernel(x, edge_index0, edge_index1, W0, al0, ar0, W1, al1, ar1,
           sw1, sb1, sw2, cw, cb):
    f32 = jnp.float32
    wcat = jnp.concatenate([W0, W1], axis=1)
    rows = jnp.arange(_D, dtype=jnp.int32)
    heads = rows // _F
    m = jnp.zeros((2 * _D, 4 * _H), f32)
    m = m.at[rows, heads].set(al0.reshape(_D))
    m = m.at[rows, _H + heads].set(ar0.reshape(_D))
    m = m.at[_D + rows, 2 * _H + heads].set(al1.reshape(_D))
    m = m.at[_D + rows, 3 * _H + heads].set(ar1.reshape(_D))

    fa0, fb0, fa1, fb1, e0, e1 = _tc_project(x, wcat, m)

    src0, dst0 = edge_index0[0], edge_index0[1]
    src1, dst1 = edge_index1[0], edge_index1[1]
    zblk = jnp.zeros((_RB, 128), f32)
    ex0, ex1, s0, s1 = _sc_softmax_denom(src0, dst0, src1, dst1, e0, e1, zblk)
    oa0, ob0, oa1, ob1 = _sc_aggregate(src0, dst0, src1, dst1,
                                       ex0, ex1, s0, s1,
                                       fa0, fb0, fa1, fb1, zblk)

    lp0, lp1, w = _tc_pool(oa0, ob0, oa1, ob1, sw1,
                           sb1.reshape(1, _HID), sw2.reshape(1, _HID), cw)
    return _tc_combine(lp0, lp1, w, cb.reshape(1, _OUT))
